# baseline (speedup 1.0000x reference)
import functools

import jax
import jax.numpy as jnp
from jax.experimental import pallas as pl
from jax.experimental.pallas import tpu as pltpu

_EPS = 1e-5


def _csum(x):
    return jnp.sum(x, axis=0, keepdims=True)


def _stage_kernel(xo_ref, xc_ref, stats_ref, xob_ref, xcb_ref):
    j = pl.program_id(0)
    xo = xo_ref[...]
    xc = xc_ref[...]
    xob_ref[...] = xo.astype(jnp.bfloat16)
    xcb_ref[...] = xc.astype(jnp.bfloat16)
    block = jnp.concatenate([
        _csum(xo), _csum(xo * xo), _csum(xc), _csum(xc * xc),
        _csum(xo * xc), jnp.zeros((3, xo.shape[1]), jnp.float32)
    ], axis=0)

    @pl.when(j == 0)
    def _():
        stats_ref[...] = block

    @pl.when(j > 0)
    def _():
        stats_ref[...] += block


def _main_kernel(inv_b,
                 xob_ref, xcb_ref, stats_ref,
                 cg1_ref, cb1_ref, cw1_ref, cc1_ref,
                 og1_ref, ob1_ref, ow1_ref, oc1_ref,
                 rg1_ref, rb1_ref, rw1_ref, rc1_ref,
                 cg2_ref, cb2_ref, cw2_ref, cc2_ref,
                 og2_ref, ob2_ref, ow2_ref, oc2_ref,
                 rg2_ref, rb2_ref, rw2_ref, rc2_ref,
                 out_c_ref, out_o_ref, out_r_ref,
                 w1big_ref, b1big_ref, w2c_ref, w2o_ref, w2r_ref, b2_ref,
                 hstats_ref):
    p = pl.program_id(0)
    j = pl.program_id(1)
    H = cw1_ref.shape[0]

    def fold(m, v, g_ref, b_ref, w_ref, c_ref):
        a = g_ref[...] * jax.lax.rsqrt(v + _EPS)
        ws = (jnp.transpose(a) * w_ref[...]).astype(jnp.bfloat16)
        bias = (jnp.dot(b_ref[...] - a * m, w_ref[...],
                        preferred_element_type=jnp.float32) + c_ref[...])
        return ws, bias

    @pl.when((p == 0) & (j == 0))
    def _():
        s = stats_ref[...]
        m_xo = s[0:1] * inv_b
        v_xo = s[1:2] * inv_b - m_xo * m_xo
        m_xc = s[2:3] * inv_b
        v_xc = s[3:4] * inv_b - m_xc * m_xc
        m_xr = m_xo + m_xc
        v_xr = (s[1:2] + s[3:4] + 2.0 * s[4:5]) * inv_b - m_xr * m_xr
        wc, bc = fold(m_xc, v_xc, cg1_ref, cb1_ref, cw1_ref, cc1_ref)
        wo, bo = fold(m_xo, v_xo, og1_ref, ob1_ref, ow1_ref, oc1_ref)
        wr, br = fold(m_xr, v_xr, rg1_ref, rb1_ref, rw1_ref, rc1_ref)
        z = jnp.zeros((H, H), jnp.bfloat16)
        w1big_ref[...] = jnp.concatenate([
            jnp.concatenate([wc, z, wr], axis=1),
            jnp.concatenate([z, wo, wr], axis=1)], axis=0)
        b1big_ref[...] = jnp.concatenate([bc, bo, br], axis=1)

    @pl.when((p == 1) & (j == 0))
    def _():
        hs = hstats_ref[...]
        for k, (g_ref, b_ref, w_ref, c_ref, w2_ref) in enumerate(
                ((cg2_ref, cb2_ref, cw2_ref, cc2_ref, w2c_ref),
                 (og2_ref, ob2_ref, ow2_ref, oc2_ref, w2o_ref),
                 (rg2_ref, rb2_ref, rw2_ref, rc2_ref, w2r_ref))):
            m = hs[0:1, k * H:(k + 1) * H] * inv_b
            v = hs[1:2, k * H:(k + 1) * H] * inv_b - m * m
            ws, bias = fold(m, v, g_ref, b_ref, w_ref, c_ref)
            w2_ref[...] = ws
            b2_ref[k:k + 1, :] = bias

    xbig = jnp.concatenate([xcb_ref[...], xob_ref[...]], axis=1)
    h = jnp.maximum(
        jnp.dot(xbig, w1big_ref[...], preferred_element_type=jnp.float32)
        + b1big_ref[...], 0.0)

    @pl.when(p == 0)
    def _():
        block = jnp.concatenate(
            [_csum(h), _csum(h * h),
             jnp.zeros((6, h.shape[1]), jnp.float32)], axis=0)

        @pl.when(j == 0)
        def _():
            hstats_ref[...] = block

        @pl.when(j > 0)
        def _():
            hstats_ref[...] += block

    @pl.when(p == 1)
    def _():
        hb = h.astype(jnp.bfloat16)

        def head(k, w2_ref):
            return (jnp.dot(hb[:, k * H:(k + 1) * H], w2_ref[...],
                            preferred_element_type=jnp.float32)
                    + b2_ref[k:k + 1, :])

        def log_softmax(z):
            m = jnp.max(z, axis=-1, keepdims=True)
            s = z - m
            return s - jnp.log(jnp.sum(jnp.exp(s), axis=-1, keepdims=True))

        out_c_ref[...] = log_softmax(head(0, w2c_ref))
        out_o_ref[...] = head(1, w2o_ref)
        out_r_ref[...] = log_softmax(head(2, w2r_ref))


def _row1(r, h):
    return pl.BlockSpec((r, h), lambda j: (j, 0))


def _vec2(h):
    return pl.BlockSpec((1, h), lambda p, j: (0, 0))


def _mat2(h, o):
    return pl.BlockSpec((h, o), lambda p, j: (0, 0))


@functools.partial(jax.jit, static_argnames=())
def kernel(xo, xc,
           ctx_g1, ctx_b1, ctx_W1, ctx_c1, ctx_g2, ctx_b2, ctx_W2, ctx_c2,
           obj_g1, obj_b1, obj_W1, obj_c1, obj_g2, obj_b2, obj_W2, obj_c2,
           rnd_g1, rnd_b1, rnd_W1, rnd_c1, rnd_g2, rnd_b2, rnd_W2, rnd_c2):
    B, H = xo.shape
    O = ctx_W2.shape[1]
    R = 2000 if B % 2000 == 0 else (1000 if B % 1000 == 0 else B)
    nb = B // R

    stats, xob, xcb = pl.pallas_call(
        _stage_kernel,
        grid=(nb,),
        in_specs=[_row1(R, H), _row1(R, H)],
        out_specs=[pl.BlockSpec((8, H), lambda j: (0, 0)),
                   _row1(R, H), _row1(R, H)],
        out_shape=[jax.ShapeDtypeStruct((8, H), jnp.float32),
                   jax.ShapeDtypeStruct((B, H), jnp.bfloat16),
                   jax.ShapeDtypeStruct((B, H), jnp.bfloat16)],
    )(xo, xc)

    vecs = {k: v.reshape(1, H) for k, v in dict(
        cg1=ctx_g1, cb1=ctx_b1, cc1=ctx_c1, og1=obj_g1, ob1=obj_b1,
        oc1=obj_c1, rg1=rnd_g1, rb1=rnd_b1, rc1=rnd_c1,
        cg2=ctx_g2, cb2=ctx_b2, cc2=ctx_c2, og2=obj_g2, ob2=obj_b2,
        oc2=obj_c2, rg2=rnd_g2, rb2=rnd_b2, rc2=rnd_c2).items()}

    Rb = 400 if B % 400 == 0 else R
    nbb = B // Rb
    row_in = pl.BlockSpec((Rb, H), lambda p, j: (j, 0))
    row_out = pl.BlockSpec((Rb, O), lambda p, j: (p * j, 0))

    outs = pl.pallas_call(
        functools.partial(_main_kernel, 1.0 / B),
        grid=(2, nbb),
        in_specs=[row_in, row_in, pl.BlockSpec((8, H), lambda p, j: (0, 0)),
                  _vec2(H), _vec2(H), _mat2(H, H), _vec2(H),
                  _vec2(H), _vec2(H), _mat2(H, H), _vec2(H),
                  _vec2(H), _vec2(H), _mat2(H, H), _vec2(H),
                  _vec2(H), _vec2(H), _mat2(H, O), _vec2(O),
                  _vec2(H), _vec2(H), _mat2(H, O), _vec2(O),
                  _vec2(H), _vec2(H), _mat2(H, O), _vec2(O)],
        out_specs=[row_out, row_out, row_out],
        out_shape=[jax.ShapeDtypeStruct((B, O), jnp.float32)] * 3,
        scratch_shapes=[pltpu.VMEM((2 * H, 3 * H), jnp.bfloat16),
                        pltpu.VMEM((1, 3 * H), jnp.float32),
                        pltpu.VMEM((H, O), jnp.bfloat16),
                        pltpu.VMEM((H, O), jnp.bfloat16),
                        pltpu.VMEM((H, O), jnp.bfloat16),
                        pltpu.VMEM((8, O), jnp.float32),
                        pltpu.VMEM((8, 3 * H), jnp.float32)],
    )(xob, xcb, stats,
      vecs["cg1"], vecs["cb1"], ctx_W1, vecs["cc1"],
      vecs["og1"], vecs["ob1"], obj_W1, vecs["oc1"],
      vecs["rg1"], vecs["rb1"], rnd_W1, vecs["rc1"],
      vecs["cg2"], vecs["cb2"], ctx_W2, vecs["cc2"],
      vecs["og2"], vecs["ob2"], obj_W2, vecs["oc2"],
      vecs["rg2"], vecs["rb2"], rnd_W2, vecs["rc2"])

    return tuple(outs)


# speedup vs baseline: 2.1981x; 2.1981x over previous
import functools

import jax
import jax.numpy as jnp
from jax.experimental import pallas as pl
from jax.experimental.pallas import tpu as pltpu

_EPS = 1e-5


def _csum(x):
    return jnp.sum(x, axis=0, keepdims=True)


def _stage_kernel(xo_ref, xc_ref, stats_ref, xob_ref, xcb_ref):
    j = pl.program_id(0)
    xo = xo_ref[...]
    xc = xc_ref[...]
    xob_ref[...] = xo.astype(jnp.bfloat16)
    xcb_ref[...] = xc.astype(jnp.bfloat16)
    block = jnp.concatenate([
        _csum(xo), _csum(xo * xo), _csum(xc), _csum(xc * xc),
        _csum(xo * xc), jnp.zeros((3, xo.shape[1]), jnp.float32)
    ], axis=0)

    @pl.when(j == 0)
    def _():
        stats_ref[...] = block

    @pl.when(j > 0)
    def _():
        stats_ref[...] += block


def _main_kernel(inv_b,
                 xob_ref, xcb_ref, stats_ref,
                 cg1_ref, cb1_ref, cw1_ref, cc1_ref,
                 og1_ref, ob1_ref, ow1_ref, oc1_ref,
                 rg1_ref, rb1_ref, rw1_ref, rc1_ref,
                 cg2_ref, cb2_ref, cw2_ref, cc2_ref,
                 og2_ref, ob2_ref, ow2_ref, oc2_ref,
                 rg2_ref, rb2_ref, rw2_ref, rc2_ref,
                 out_c_ref, out_o_ref, out_r_ref,
                 w1big_ref, b1big_ref, w2c_ref, w2o_ref, w2r_ref, b2_ref,
                 hstats_ref):
    p = pl.program_id(0)
    j = pl.program_id(1)
    H = cw1_ref.shape[0]

    def fold(m, v, g_ref, b_ref, w_ref, c_ref):
        a = g_ref[...] * jax.lax.rsqrt(v + _EPS)
        ws = (jnp.transpose(a) * w_ref[...]).astype(jnp.bfloat16)
        bias = (jnp.dot(b_ref[...] - a * m, w_ref[...],
                        preferred_element_type=jnp.float32) + c_ref[...])
        return ws, bias

    @pl.when((p == 0) & (j == 0))
    def _():
        s = stats_ref[...]
        m_xo = s[0:1] * inv_b
        v_xo = s[1:2] * inv_b - m_xo * m_xo
        m_xc = s[2:3] * inv_b
        v_xc = s[3:4] * inv_b - m_xc * m_xc
        m_xr = m_xo + m_xc
        v_xr = (s[1:2] + s[3:4] + 2.0 * s[4:5]) * inv_b - m_xr * m_xr
        wc, bc = fold(m_xc, v_xc, cg1_ref, cb1_ref, cw1_ref, cc1_ref)
        wo, bo = fold(m_xo, v_xo, og1_ref, ob1_ref, ow1_ref, oc1_ref)
        wr, br = fold(m_xr, v_xr, rg1_ref, rb1_ref, rw1_ref, rc1_ref)
        z = jnp.zeros((H, H), jnp.bfloat16)
        w1big_ref[...] = jnp.concatenate([
            jnp.concatenate([wc, z, wr], axis=1),
            jnp.concatenate([z, wo, wr], axis=1)], axis=0)
        b1big_ref[...] = jnp.concatenate([bc, bo, br], axis=1)

    @pl.when((p == 1) & (j == 0))
    def _():
        hs = hstats_ref[...]
        for k, (g_ref, b_ref, w_ref, c_ref, w2_ref) in enumerate(
                ((cg2_ref, cb2_ref, cw2_ref, cc2_ref, w2c_ref),
                 (og2_ref, ob2_ref, ow2_ref, oc2_ref, w2o_ref),
                 (rg2_ref, rb2_ref, rw2_ref, rc2_ref, w2r_ref))):
            m = hs[0:1, k * H:(k + 1) * H] * inv_b
            v = hs[1:2, k * H:(k + 1) * H] * inv_b - m * m
            ws, bias = fold(m, v, g_ref, b_ref, w_ref, c_ref)
            w2_ref[...] = ws
            b2_ref[k:k + 1, :] = bias

    xbig = jnp.concatenate([xcb_ref[...], xob_ref[...]], axis=1)
    h = jnp.maximum(
        jnp.dot(xbig, w1big_ref[...], preferred_element_type=jnp.float32)
        + b1big_ref[...], 0.0)

    @pl.when(p == 0)
    def _():
        block = jnp.concatenate(
            [_csum(h), _csum(h * h),
             jnp.zeros((6, h.shape[1]), jnp.float32)], axis=0)

        @pl.when(j == 0)
        def _():
            hstats_ref[...] = block

        @pl.when(j > 0)
        def _():
            hstats_ref[...] += block

    @pl.when(p == 1)
    def _():
        hb = h.astype(jnp.bfloat16)

        def head(k, w2_ref):
            return (jnp.dot(hb[:, k * H:(k + 1) * H], w2_ref[...],
                            preferred_element_type=jnp.float32)
                    + b2_ref[k:k + 1, :])

        def log_softmax(z):
            m = jnp.max(z, axis=-1, keepdims=True)
            s = z - m
            return s - jnp.log(jnp.sum(jnp.exp(s), axis=-1, keepdims=True))

        out_c_ref[...] = log_softmax(head(0, w2c_ref))
        out_o_ref[...] = head(1, w2o_ref)
        out_r_ref[...] = log_softmax(head(2, w2r_ref))


def _row1(r, h):
    return pl.BlockSpec((r, h), lambda j: (j, 0))


def _vec2(h):
    return pl.BlockSpec((1, h), lambda p, j: (0, 0))


def _mat2(h, o):
    return pl.BlockSpec((h, o), lambda p, j: (0, 0))


@functools.partial(jax.jit, static_argnames=())
def kernel(xo, xc,
           ctx_g1, ctx_b1, ctx_W1, ctx_c1, ctx_g2, ctx_b2, ctx_W2, ctx_c2,
           obj_g1, obj_b1, obj_W1, obj_c1, obj_g2, obj_b2, obj_W2, obj_c2,
           rnd_g1, rnd_b1, rnd_W1, rnd_c1, rnd_g2, rnd_b2, rnd_W2, rnd_c2):
    B, H = xo.shape
    O = ctx_W2.shape[1]
    R = 2000 if B % 2000 == 0 else (1000 if B % 1000 == 0 else B)
    nb = B // R

    stats, xob, xcb = pl.pallas_call(
        _stage_kernel,
        grid=(nb,),
        in_specs=[_row1(R, H), _row1(R, H)],
        out_specs=[pl.BlockSpec((8, H), lambda j: (0, 0)),
                   _row1(R, H), _row1(R, H)],
        out_shape=[jax.ShapeDtypeStruct((8, H), jnp.float32),
                   jax.ShapeDtypeStruct((B, H), jnp.bfloat16),
                   jax.ShapeDtypeStruct((B, H), jnp.bfloat16)],
    )(xo, xc)

    vecs = {k: v.reshape(1, H) for k, v in dict(
        cg1=ctx_g1, cb1=ctx_b1, cc1=ctx_c1, og1=obj_g1, ob1=obj_b1,
        oc1=obj_c1, rg1=rnd_g1, rb1=rnd_b1, rc1=rnd_c1,
        cg2=ctx_g2, cb2=ctx_b2, cc2=ctx_c2, og2=obj_g2, ob2=obj_b2,
        oc2=obj_c2, rg2=rnd_g2, rb2=rnd_b2, rc2=rnd_c2).items()}

    Rb = 4000 if B % 4000 == 0 else R
    nbb = B // Rb
    row_in = pl.BlockSpec((Rb, H), lambda p, j: (j, 0))
    row_out = pl.BlockSpec((Rb, O), lambda p, j: (p * j, 0))

    outs = pl.pallas_call(
        functools.partial(_main_kernel, 1.0 / B),
        grid=(2, nbb),
        in_specs=[row_in, row_in, pl.BlockSpec((8, H), lambda p, j: (0, 0)),
                  _vec2(H), _vec2(H), _mat2(H, H), _vec2(H),
                  _vec2(H), _vec2(H), _mat2(H, H), _vec2(H),
                  _vec2(H), _vec2(H), _mat2(H, H), _vec2(H),
                  _vec2(H), _vec2(H), _mat2(H, O), _vec2(O),
                  _vec2(H), _vec2(H), _mat2(H, O), _vec2(O),
                  _vec2(H), _vec2(H), _mat2(H, O), _vec2(O)],
        out_specs=[row_out, row_out, row_out],
        out_shape=[jax.ShapeDtypeStruct((B, O), jnp.float32)] * 3,
        scratch_shapes=[pltpu.VMEM((2 * H, 3 * H), jnp.bfloat16),
                        pltpu.VMEM((1, 3 * H), jnp.float32),
                        pltpu.VMEM((H, O), jnp.bfloat16),
                        pltpu.VMEM((H, O), jnp.bfloat16),
                        pltpu.VMEM((H, O), jnp.bfloat16),
                        pltpu.VMEM((8, O), jnp.float32),
                        pltpu.VMEM((8, 3 * H), jnp.float32)],
    )(xob, xcb, stats,
      vecs["cg1"], vecs["cb1"], ctx_W1, vecs["cc1"],
      vecs["og1"], vecs["ob1"], obj_W1, vecs["oc1"],
      vecs["rg1"], vecs["rb1"], rnd_W1, vecs["rc1"],
      vecs["cg2"], vecs["cb2"], ctx_W2, vecs["cc2"],
      vecs["og2"], vecs["ob2"], obj_W2, vecs["oc2"],
      vecs["rg2"], vecs["rb2"], rnd_W2, vecs["rc2"])

    return tuple(outs)


# speedup vs baseline: 2.4483x; 1.1139x over previous
import functools

import jax
import jax.numpy as jnp
from jax.experimental import pallas as pl
from jax.experimental.pallas import tpu as pltpu

_EPS = 1e-5


def _csum(x):
    return jnp.sum(x, axis=0, keepdims=True)


def _stage_kernel(xo_ref, xc_ref, stats_ref, xob_ref, xcb_ref):
    j = pl.program_id(0)
    xo = xo_ref[...]
    xc = xc_ref[...]
    xob_ref[...] = xo.astype(jnp.bfloat16)
    xcb_ref[...] = xc.astype(jnp.bfloat16)
    block = jnp.concatenate([
        _csum(xo), _csum(xo * xo), _csum(xc), _csum(xc * xc),
        _csum(xo * xc), jnp.zeros((3, xo.shape[1]), jnp.float32)
    ], axis=0)

    @pl.when(j == 0)
    def _():
        stats_ref[...] = block

    @pl.when(j > 0)
    def _():
        stats_ref[...] += block


def _main_kernel(inv_b,
                 xob_ref, xcb_ref, stats_ref,
                 cg1_ref, cb1_ref, cw1_ref, cc1_ref,
                 og1_ref, ob1_ref, ow1_ref, oc1_ref,
                 rg1_ref, rb1_ref, rw1_ref, rc1_ref,
                 cg2_ref, cb2_ref, cw2_ref, cc2_ref,
                 og2_ref, ob2_ref, ow2_ref, oc2_ref,
                 rg2_ref, rb2_ref, rw2_ref, rc2_ref,
                 out_c_ref, out_o_ref, out_r_ref,
                 w1big_ref, b1big_ref, w2c_ref, w2o_ref, w2r_ref, b2_ref,
                 hstats_ref):
    p = pl.program_id(0)
    j = pl.program_id(1)
    H = cw1_ref.shape[0]

    def fold(m, v, g_ref, b_ref, w_ref, c_ref):
        a = g_ref[...] * jax.lax.rsqrt(v + _EPS)
        ws = (jnp.transpose(a) * w_ref[...]).astype(jnp.bfloat16)
        bias = (jnp.dot(b_ref[...] - a * m, w_ref[...],
                        preferred_element_type=jnp.float32) + c_ref[...])
        return ws, bias

    @pl.when((p == 0) & (j == 0))
    def _():
        s = stats_ref[...]
        m_xo = s[0:1] * inv_b
        v_xo = s[1:2] * inv_b - m_xo * m_xo
        m_xc = s[2:3] * inv_b
        v_xc = s[3:4] * inv_b - m_xc * m_xc
        m_xr = m_xo + m_xc
        v_xr = (s[1:2] + s[3:4] + 2.0 * s[4:5]) * inv_b - m_xr * m_xr
        wc, bc = fold(m_xc, v_xc, cg1_ref, cb1_ref, cw1_ref, cc1_ref)
        wo, bo = fold(m_xo, v_xo, og1_ref, ob1_ref, ow1_ref, oc1_ref)
        wr, br = fold(m_xr, v_xr, rg1_ref, rb1_ref, rw1_ref, rc1_ref)
        z = jnp.zeros((H, H), jnp.bfloat16)
        w1big_ref[...] = jnp.concatenate([
            jnp.concatenate([wc, z, wr], axis=1),
            jnp.concatenate([z, wo, wr], axis=1)], axis=0)
        b1big_ref[...] = jnp.concatenate([bc, bo, br], axis=1)

    @pl.when((p == 1) & (j == 0))
    def _():
        hs = hstats_ref[...]
        for k, (g_ref, b_ref, w_ref, c_ref, w2_ref) in enumerate(
                ((cg2_ref, cb2_ref, cw2_ref, cc2_ref, w2c_ref),
                 (og2_ref, ob2_ref, ow2_ref, oc2_ref, w2o_ref),
                 (rg2_ref, rb2_ref, rw2_ref, rc2_ref, w2r_ref))):
            m = hs[0:1, k * H:(k + 1) * H] * inv_b
            v = hs[1:2, k * H:(k + 1) * H] * inv_b - m * m
            ws, bias = fold(m, v, g_ref, b_ref, w_ref, c_ref)
            w2_ref[...] = ws
            b2_ref[k:k + 1, :] = bias

    xbig = jnp.concatenate([xcb_ref[...], xob_ref[...]], axis=1)
    h = jnp.maximum(
        jnp.dot(xbig, w1big_ref[...], preferred_element_type=jnp.float32)
        + b1big_ref[...], 0.0)

    @pl.when(p == 0)
    def _():
        block = jnp.concatenate(
            [_csum(h), _csum(h * h),
             jnp.zeros((6, h.shape[1]), jnp.float32)], axis=0)

        @pl.when(j == 0)
        def _():
            hstats_ref[...] = block

        @pl.when(j > 0)
        def _():
            hstats_ref[...] += block

    @pl.when(p == 1)
    def _():
        hb = h.astype(jnp.bfloat16)

        def head(k, w2_ref):
            return (jnp.dot(hb[:, k * H:(k + 1) * H], w2_ref[...],
                            preferred_element_type=jnp.float32)
                    + b2_ref[k:k + 1, :])

        def log_softmax(z):
            m = jnp.max(z, axis=-1, keepdims=True)
            s = z - m
            return s - jnp.log(jnp.sum(jnp.exp(s), axis=-1, keepdims=True))

        out_c_ref[...] = log_softmax(head(0, w2c_ref))
        out_o_ref[...] = head(1, w2o_ref)
        out_r_ref[...] = log_softmax(head(2, w2r_ref))


def _row1(r, h):
    return pl.BlockSpec((r, h), lambda j: (j, 0))


def _vec2(h):
    return pl.BlockSpec((1, h), lambda p, j: (0, 0))


def _mat2(h, o):
    return pl.BlockSpec((h, o), lambda p, j: (0, 0))


@functools.partial(jax.jit, static_argnames=())
def kernel(xo, xc,
           ctx_g1, ctx_b1, ctx_W1, ctx_c1, ctx_g2, ctx_b2, ctx_W2, ctx_c2,
           obj_g1, obj_b1, obj_W1, obj_c1, obj_g2, obj_b2, obj_W2, obj_c2,
           rnd_g1, rnd_b1, rnd_W1, rnd_c1, rnd_g2, rnd_b2, rnd_W2, rnd_c2):
    B, H = xo.shape
    O = ctx_W2.shape[1]
    R = 5000 if B % 5000 == 0 else (1000 if B % 1000 == 0 else B)
    nb = B // R

    stats, xob, xcb = pl.pallas_call(
        _stage_kernel,
        grid=(nb,),
        in_specs=[_row1(R, H), _row1(R, H)],
        out_specs=[pl.BlockSpec((8, H), lambda j: (0, 0)),
                   _row1(R, H), _row1(R, H)],
        out_shape=[jax.ShapeDtypeStruct((8, H), jnp.float32),
                   jax.ShapeDtypeStruct((B, H), jnp.bfloat16),
                   jax.ShapeDtypeStruct((B, H), jnp.bfloat16)],
    )(xo, xc)

    vecs = {k: v.reshape(1, H) for k, v in dict(
        cg1=ctx_g1, cb1=ctx_b1, cc1=ctx_c1, og1=obj_g1, ob1=obj_b1,
        oc1=obj_c1, rg1=rnd_g1, rb1=rnd_b1, rc1=rnd_c1,
        cg2=ctx_g2, cb2=ctx_b2, cc2=ctx_c2, og2=obj_g2, ob2=obj_b2,
        oc2=obj_c2, rg2=rnd_g2, rb2=rnd_b2, rc2=rnd_c2).items()}

    Rb = 5000 if B % 5000 == 0 else R
    nbb = B // Rb
    row_in = pl.BlockSpec((Rb, H), lambda p, j: (j, 0))
    row_out = pl.BlockSpec((Rb, O), lambda p, j: (p * j, 0))

    outs = pl.pallas_call(
        functools.partial(_main_kernel, 1.0 / B),
        grid=(2, nbb),
        in_specs=[row_in, row_in, pl.BlockSpec((8, H), lambda p, j: (0, 0)),
                  _vec2(H), _vec2(H), _mat2(H, H), _vec2(H),
                  _vec2(H), _vec2(H), _mat2(H, H), _vec2(H),
                  _vec2(H), _vec2(H), _mat2(H, H), _vec2(H),
                  _vec2(H), _vec2(H), _mat2(H, O), _vec2(O),
                  _vec2(H), _vec2(H), _mat2(H, O), _vec2(O),
                  _vec2(H), _vec2(H), _mat2(H, O), _vec2(O)],
        out_specs=[row_out, row_out, row_out],
        out_shape=[jax.ShapeDtypeStruct((B, O), jnp.float32)] * 3,
        scratch_shapes=[pltpu.VMEM((2 * H, 3 * H), jnp.bfloat16),
                        pltpu.VMEM((1, 3 * H), jnp.float32),
                        pltpu.VMEM((H, O), jnp.bfloat16),
                        pltpu.VMEM((H, O), jnp.bfloat16),
                        pltpu.VMEM((H, O), jnp.bfloat16),
                        pltpu.VMEM((8, O), jnp.float32),
                        pltpu.VMEM((8, 3 * H), jnp.float32)],
    )(xob, xcb, stats,
      vecs["cg1"], vecs["cb1"], ctx_W1, vecs["cc1"],
      vecs["og1"], vecs["ob1"], obj_W1, vecs["oc1"],
      vecs["rg1"], vecs["rb1"], rnd_W1, vecs["rc1"],
      vecs["cg2"], vecs["cb2"], ctx_W2, vecs["cc2"],
      vecs["og2"], vecs["ob2"], obj_W2, vecs["oc2"],
      vecs["rg2"], vecs["rb2"], rnd_W2, vecs["rc2"])

    return tuple(outs)


# speedup vs baseline: 2.4762x; 1.0114x over previous
import functools

import jax
import jax.numpy as jnp
from jax.experimental import pallas as pl
from jax.experimental.pallas import tpu as pltpu

_EPS = 1e-5


def _csum(x):
    return jnp.sum(x, axis=0, keepdims=True)


def _stage_kernel(xo_ref, xc_ref, stats_ref, xob_ref, xcb_ref):
    j = pl.program_id(0)
    xo = xo_ref[...]
    xc = xc_ref[...]
    xob_ref[...] = xo.astype(jnp.bfloat16)
    xcb_ref[...] = xc.astype(jnp.bfloat16)
    block = jnp.concatenate([
        _csum(xo), _csum(xo * xo), _csum(xc), _csum(xc * xc),
        _csum(xo * xc), jnp.zeros((3, xo.shape[1]), jnp.float32)
    ], axis=0)

    @pl.when(j == 0)
    def _():
        stats_ref[...] = block

    @pl.when(j > 0)
    def _():
        stats_ref[...] += block


def _main_kernel(inv_b,
                 xob_ref, xcb_ref, stats_ref,
                 cg1_ref, cb1_ref, cw1_ref, cc1_ref,
                 og1_ref, ob1_ref, ow1_ref, oc1_ref,
                 rg1_ref, rb1_ref, rw1_ref, rc1_ref,
                 cg2_ref, cb2_ref, cw2_ref, cc2_ref,
                 og2_ref, ob2_ref, ow2_ref, oc2_ref,
                 rg2_ref, rb2_ref, rw2_ref, rc2_ref,
                 out_c_ref, out_o_ref, out_r_ref,
                 w1big_ref, b1big_ref, w2c_ref, w2o_ref, w2r_ref, b2_ref,
                 hstats_ref):
    p = pl.program_id(0)
    j = pl.program_id(1)
    H = cw1_ref.shape[0]

    def fold(m, v, g_ref, b_ref, w_ref, c_ref):
        a = g_ref[...] * jax.lax.rsqrt(v + _EPS)
        ws = (jnp.transpose(a) * w_ref[...]).astype(jnp.bfloat16)
        bias = (jnp.dot(b_ref[...] - a * m, w_ref[...],
                        preferred_element_type=jnp.float32) + c_ref[...])
        return ws, bias

    @pl.when((p == 0) & (j == 0))
    def _():
        s = stats_ref[...]
        m_xo = s[0:1] * inv_b
        v_xo = s[1:2] * inv_b - m_xo * m_xo
        m_xc = s[2:3] * inv_b
        v_xc = s[3:4] * inv_b - m_xc * m_xc
        m_xr = m_xo + m_xc
        v_xr = (s[1:2] + s[3:4] + 2.0 * s[4:5]) * inv_b - m_xr * m_xr
        wc, bc = fold(m_xc, v_xc, cg1_ref, cb1_ref, cw1_ref, cc1_ref)
        wo, bo = fold(m_xo, v_xo, og1_ref, ob1_ref, ow1_ref, oc1_ref)
        wr, br = fold(m_xr, v_xr, rg1_ref, rb1_ref, rw1_ref, rc1_ref)
        z = jnp.zeros((H, H), jnp.bfloat16)
        w1big_ref[...] = jnp.concatenate([
            jnp.concatenate([wc, z, wr], axis=1),
            jnp.concatenate([z, wo, wr], axis=1)], axis=0)
        b1big_ref[...] = jnp.concatenate([bc, bo, br], axis=1)

    @pl.when((p == 1) & (j == 0))
    def _():
        hs = hstats_ref[...]
        for k, (g_ref, b_ref, w_ref, c_ref, w2_ref) in enumerate(
                ((cg2_ref, cb2_ref, cw2_ref, cc2_ref, w2c_ref),
                 (og2_ref, ob2_ref, ow2_ref, oc2_ref, w2o_ref),
                 (rg2_ref, rb2_ref, rw2_ref, rc2_ref, w2r_ref))):
            m = hs[0:1, k * H:(k + 1) * H] * inv_b
            v = hs[1:2, k * H:(k + 1) * H] * inv_b - m * m
            ws, bias = fold(m, v, g_ref, b_ref, w_ref, c_ref)
            w2_ref[...] = ws
            b2_ref[k:k + 1, :] = bias

    xbig = jnp.concatenate([xcb_ref[...], xob_ref[...]], axis=1)
    h = jnp.maximum(
        jnp.dot(xbig, w1big_ref[...], preferred_element_type=jnp.float32)
        + b1big_ref[...], 0.0)

    @pl.when(p == 0)
    def _():
        block = jnp.concatenate(
            [_csum(h), _csum(h * h),
             jnp.zeros((6, h.shape[1]), jnp.float32)], axis=0)

        @pl.when(j == 0)
        def _():
            hstats_ref[...] = block

        @pl.when(j > 0)
        def _():
            hstats_ref[...] += block

    @pl.when(p == 1)
    def _():
        hb = h.astype(jnp.bfloat16)

        def head(k, w2_ref):
            return (jnp.dot(hb[:, k * H:(k + 1) * H], w2_ref[...],
                            preferred_element_type=jnp.float32)
                    + b2_ref[k:k + 1, :])

        def log_softmax(z):
            m = jnp.max(z, axis=-1, keepdims=True)
            s = z - m
            return s - jnp.log(jnp.sum(jnp.exp(s), axis=-1, keepdims=True))

        out_c_ref[...] = log_softmax(head(0, w2c_ref))
        out_o_ref[...] = head(1, w2o_ref)
        out_r_ref[...] = log_softmax(head(2, w2r_ref))


def _row1(r, h):
    return pl.BlockSpec((r, h), lambda j: (j, 0))


def _vec2(h):
    return pl.BlockSpec((1, h), lambda p, j: (0, 0))


def _mat2(h, o):
    return pl.BlockSpec((h, o), lambda p, j: (0, 0))


@functools.partial(jax.jit, static_argnames=())
def kernel(xo, xc,
           ctx_g1, ctx_b1, ctx_W1, ctx_c1, ctx_g2, ctx_b2, ctx_W2, ctx_c2,
           obj_g1, obj_b1, obj_W1, obj_c1, obj_g2, obj_b2, obj_W2, obj_c2,
           rnd_g1, rnd_b1, rnd_W1, rnd_c1, rnd_g2, rnd_b2, rnd_W2, rnd_c2):
    B, H = xo.shape
    O = ctx_W2.shape[1]
    R = 10000 if B % 10000 == 0 else (1000 if B % 1000 == 0 else B)
    nb = B // R

    stats, xob, xcb = pl.pallas_call(
        _stage_kernel,
        grid=(nb,),
        in_specs=[_row1(R, H), _row1(R, H)],
        out_specs=[pl.BlockSpec((8, H), lambda j: (0, 0)),
                   _row1(R, H), _row1(R, H)],
        out_shape=[jax.ShapeDtypeStruct((8, H), jnp.float32),
                   jax.ShapeDtypeStruct((B, H), jnp.bfloat16),
                   jax.ShapeDtypeStruct((B, H), jnp.bfloat16)],
    )(xo, xc)

    vecs = {k: v.reshape(1, H) for k, v in dict(
        cg1=ctx_g1, cb1=ctx_b1, cc1=ctx_c1, og1=obj_g1, ob1=obj_b1,
        oc1=obj_c1, rg1=rnd_g1, rb1=rnd_b1, rc1=rnd_c1,
        cg2=ctx_g2, cb2=ctx_b2, cc2=ctx_c2, og2=obj_g2, ob2=obj_b2,
        oc2=obj_c2, rg2=rnd_g2, rb2=rnd_b2, rc2=rnd_c2).items()}

    Rb = 5000 if B % 5000 == 0 else R
    nbb = B // Rb
    row_in = pl.BlockSpec((Rb, H), lambda p, j: (j, 0))
    row_out = pl.BlockSpec((Rb, O), lambda p, j: (p * j, 0))

    outs = pl.pallas_call(
        functools.partial(_main_kernel, 1.0 / B),
        grid=(2, nbb),
        in_specs=[row_in, row_in, pl.BlockSpec((8, H), lambda p, j: (0, 0)),
                  _vec2(H), _vec2(H), _mat2(H, H), _vec2(H),
                  _vec2(H), _vec2(H), _mat2(H, H), _vec2(H),
                  _vec2(H), _vec2(H), _mat2(H, H), _vec2(H),
                  _vec2(H), _vec2(H), _mat2(H, O), _vec2(O),
                  _vec2(H), _vec2(H), _mat2(H, O), _vec2(O),
                  _vec2(H), _vec2(H), _mat2(H, O), _vec2(O)],
        out_specs=[row_out, row_out, row_out],
        out_shape=[jax.ShapeDtypeStruct((B, O), jnp.float32)] * 3,
        scratch_shapes=[pltpu.VMEM((2 * H, 3 * H), jnp.bfloat16),
                        pltpu.VMEM((1, 3 * H), jnp.float32),
                        pltpu.VMEM((H, O), jnp.bfloat16),
                        pltpu.VMEM((H, O), jnp.bfloat16),
                        pltpu.VMEM((H, O), jnp.bfloat16),
                        pltpu.VMEM((8, O), jnp.float32),
                        pltpu.VMEM((8, 3 * H), jnp.float32)],
    )(xob, xcb, stats,
      vecs["cg1"], vecs["cb1"], ctx_W1, vecs["cc1"],
      vecs["og1"], vecs["ob1"], obj_W1, vecs["oc1"],
      vecs["rg1"], vecs["rb1"], rnd_W1, vecs["rc1"],
      vecs["cg2"], vecs["cb2"], ctx_W2, vecs["cc2"],
      vecs["og2"], vecs["ob2"], obj_W2, vecs["oc2"],
      vecs["rg2"], vecs["rb2"], rnd_W2, vecs["rc2"])

    return tuple(outs)


# speedup vs baseline: 2.4887x; 1.0051x over previous
import functools

import jax
import jax.numpy as jnp
from jax.experimental import pallas as pl
from jax.experimental.pallas import tpu as pltpu

_EPS = 1e-5


def _csum(x):
    return jnp.sum(x, axis=0, keepdims=True)


def _stage_kernel(xo_ref, xc_ref, stats_ref, xb_ref):
    j = pl.program_id(0)
    xo = xo_ref[...]
    xc = xc_ref[...]
    xb_ref[...] = jnp.concatenate(
        [xc.astype(jnp.bfloat16), xo.astype(jnp.bfloat16)], axis=1)
    block = jnp.concatenate([
        _csum(xo), _csum(xo * xo), _csum(xc), _csum(xc * xc),
        _csum(xo * xc), jnp.zeros((3, xo.shape[1]), jnp.float32)
    ], axis=0)

    @pl.when(j == 0)
    def _():
        stats_ref[...] = block

    @pl.when(j > 0)
    def _():
        stats_ref[...] += block


def _main_kernel(inv_b,
                 xb_ref, stats_ref,
                 cg1_ref, cb1_ref, cw1_ref, cc1_ref,
                 og1_ref, ob1_ref, ow1_ref, oc1_ref,
                 rg1_ref, rb1_ref, rw1_ref, rc1_ref,
                 cg2_ref, cb2_ref, cw2_ref, cc2_ref,
                 og2_ref, ob2_ref, ow2_ref, oc2_ref,
                 rg2_ref, rb2_ref, rw2_ref, rc2_ref,
                 out_c_ref, out_o_ref, out_r_ref,
                 w1big_ref, b1big_ref, w2co_ref, w2r_ref, b2_ref,
                 hstats_ref):
    p = pl.program_id(0)
    j = pl.program_id(1)
    H = cw1_ref.shape[0]

    def fold(m, v, g_ref, b_ref, w_ref, c_ref):
        a = g_ref[...] * jax.lax.rsqrt(v + _EPS)
        ws = (jnp.transpose(a) * w_ref[...]).astype(jnp.bfloat16)
        bias = (jnp.dot(b_ref[...] - a * m, w_ref[...],
                        preferred_element_type=jnp.float32) + c_ref[...])
        return ws, bias

    @pl.when((p == 0) & (j == 0))
    def _():
        s = stats_ref[...]
        m_xo = s[0:1] * inv_b
        v_xo = s[1:2] * inv_b - m_xo * m_xo
        m_xc = s[2:3] * inv_b
        v_xc = s[3:4] * inv_b - m_xc * m_xc
        m_xr = m_xo + m_xc
        v_xr = (s[1:2] + s[3:4] + 2.0 * s[4:5]) * inv_b - m_xr * m_xr
        wc, bc = fold(m_xc, v_xc, cg1_ref, cb1_ref, cw1_ref, cc1_ref)
        wo, bo = fold(m_xo, v_xo, og1_ref, ob1_ref, ow1_ref, oc1_ref)
        wr, br = fold(m_xr, v_xr, rg1_ref, rb1_ref, rw1_ref, rc1_ref)
        z = jnp.zeros((H, H), jnp.bfloat16)
        w1big_ref[...] = jnp.concatenate([
            jnp.concatenate([wc, z, wr], axis=1),
            jnp.concatenate([z, wo, wr], axis=1)], axis=0)
        b1big_ref[...] = jnp.concatenate([bc, bo, br], axis=1)

    @pl.when((p == 1) & (j == 0))
    def _():
        hs = hstats_ref[...]
        folded = []
        for k, (g_ref, b_ref, w_ref, c_ref) in enumerate(
                ((cg2_ref, cb2_ref, cw2_ref, cc2_ref),
                 (og2_ref, ob2_ref, ow2_ref, oc2_ref),
                 (rg2_ref, rb2_ref, rw2_ref, rc2_ref))):
            m = hs[0:1, k * H:(k + 1) * H] * inv_b
            v = hs[1:2, k * H:(k + 1) * H] * inv_b - m * m
            folded.append(fold(m, v, g_ref, b_ref, w_ref, c_ref))
        (wsc, bc), (wso, bo), (wsr, br) = folded
        z = jnp.zeros((H, H), jnp.bfloat16)
        w2co_ref[...] = jnp.concatenate([
            jnp.concatenate([wsc, z], axis=1),
            jnp.concatenate([z, wso], axis=1)], axis=0)
        w2r_ref[...] = wsr
        b2_ref[0:1, :] = jnp.concatenate([bc, bo], axis=1)
        b2_ref[1:2, 0:H] = br

    xbig = xb_ref[...]
    h = jnp.maximum(
        jnp.dot(xbig, w1big_ref[...], preferred_element_type=jnp.float32)
        + b1big_ref[...], 0.0)

    @pl.when(p == 0)
    def _():
        block = jnp.concatenate(
            [_csum(h), _csum(h * h),
             jnp.zeros((6, h.shape[1]), jnp.float32)], axis=0)

        @pl.when(j == 0)
        def _():
            hstats_ref[...] = block

        @pl.when(j > 0)
        def _():
            hstats_ref[...] += block

    @pl.when(p == 1)
    def _():
        hb = h.astype(jnp.bfloat16)

        def log_softmax(z):
            m = jnp.max(z, axis=-1, keepdims=True)
            s = z - m
            return s - jnp.log(jnp.sum(jnp.exp(s), axis=-1, keepdims=True))

        z_co = (jnp.dot(hb[:, 0:2 * H], w2co_ref[...],
                        preferred_element_type=jnp.float32)
                + b2_ref[0:1, :])
        z_r = (jnp.dot(hb[:, 2 * H:3 * H], w2r_ref[...],
                       preferred_element_type=jnp.float32)
               + b2_ref[1:2, 0:H])
        out_c_ref[...] = log_softmax(z_co[:, 0:H])
        out_o_ref[...] = z_co[:, H:2 * H]
        out_r_ref[...] = log_softmax(z_r)


def _row1(r, h):
    return pl.BlockSpec((r, h), lambda j: (j, 0))


def _vec2(h):
    return pl.BlockSpec((1, h), lambda p, j: (0, 0))


def _mat2(h, o):
    return pl.BlockSpec((h, o), lambda p, j: (0, 0))


@functools.partial(jax.jit, static_argnames=())
def kernel(xo, xc,
           ctx_g1, ctx_b1, ctx_W1, ctx_c1, ctx_g2, ctx_b2, ctx_W2, ctx_c2,
           obj_g1, obj_b1, obj_W1, obj_c1, obj_g2, obj_b2, obj_W2, obj_c2,
           rnd_g1, rnd_b1, rnd_W1, rnd_c1, rnd_g2, rnd_b2, rnd_W2, rnd_c2):
    B, H = xo.shape
    O = ctx_W2.shape[1]
    R = 10000 if B % 10000 == 0 else (1000 if B % 1000 == 0 else B)
    nb = B // R

    stats, xb = pl.pallas_call(
        _stage_kernel,
        grid=(nb,),
        in_specs=[_row1(R, H), _row1(R, H)],
        out_specs=[pl.BlockSpec((8, H), lambda j: (0, 0)),
                   _row1(R, 2 * H)],
        out_shape=[jax.ShapeDtypeStruct((8, H), jnp.float32),
                   jax.ShapeDtypeStruct((B, 2 * H), jnp.bfloat16)],
    )(xo, xc)

    vecs = {k: v.reshape(1, H) for k, v in dict(
        cg1=ctx_g1, cb1=ctx_b1, cc1=ctx_c1, og1=obj_g1, ob1=obj_b1,
        oc1=obj_c1, rg1=rnd_g1, rb1=rnd_b1, rc1=rnd_c1,
        cg2=ctx_g2, cb2=ctx_b2, cc2=ctx_c2, og2=obj_g2, ob2=obj_b2,
        oc2=obj_c2, rg2=rnd_g2, rb2=rnd_b2, rc2=rnd_c2).items()}

    Rb = 5000 if B % 5000 == 0 else R
    nbb = B // Rb
    row_in = pl.BlockSpec((Rb, 2 * H), lambda p, j: (j, 0))
    row_out = pl.BlockSpec((Rb, O), lambda p, j: (p * j, 0))

    outs = pl.pallas_call(
        functools.partial(_main_kernel, 1.0 / B),
        grid=(2, nbb),
        in_specs=[row_in, pl.BlockSpec((8, H), lambda p, j: (0, 0)),
                  _vec2(H), _vec2(H), _mat2(H, H), _vec2(H),
                  _vec2(H), _vec2(H), _mat2(H, H), _vec2(H),
                  _vec2(H), _vec2(H), _mat2(H, H), _vec2(H),
                  _vec2(H), _vec2(H), _mat2(H, O), _vec2(O),
                  _vec2(H), _vec2(H), _mat2(H, O), _vec2(O),
                  _vec2(H), _vec2(H), _mat2(H, O), _vec2(O)],
        out_specs=[row_out, row_out, row_out],
        out_shape=[jax.ShapeDtypeStruct((B, O), jnp.float32)] * 3,
        scratch_shapes=[pltpu.VMEM((2 * H, 3 * H), jnp.bfloat16),
                        pltpu.VMEM((1, 3 * H), jnp.float32),
                        pltpu.VMEM((2 * H, 2 * H), jnp.bfloat16),
                        pltpu.VMEM((H, O), jnp.bfloat16),
                        pltpu.VMEM((8, 2 * H), jnp.float32),
                        pltpu.VMEM((8, 3 * H), jnp.float32)],
    )(xb, stats,
      vecs["cg1"], vecs["cb1"], ctx_W1, vecs["cc1"],
      vecs["og1"], vecs["ob1"], obj_W1, vecs["oc1"],
      vecs["rg1"], vecs["rb1"], rnd_W1, vecs["rc1"],
      vecs["cg2"], vecs["cb2"], ctx_W2, vecs["cc2"],
      vecs["og2"], vecs["ob2"], obj_W2, vecs["oc2"],
      vecs["rg2"], vecs["rb2"], rnd_W2, vecs["rc2"])

    return tuple(outs)


# speedup vs baseline: 3.1709x; 1.2741x over previous
import functools

import jax
import jax.numpy as jnp
from jax.experimental import pallas as pl
from jax.experimental.pallas import tpu as pltpu

_EPS = 1e-5
_NSUB = 2


def _csum(x):
    return jnp.sum(x, axis=0, keepdims=True)


def _stage_kernel(xo_ref, xc_ref, stats_ref, xb_ref):
    j = pl.program_id(0)
    xo = xo_ref[...]
    xc = xc_ref[...]
    xb_ref[...] = jnp.concatenate(
        [xc.astype(jnp.bfloat16), xo.astype(jnp.bfloat16)], axis=1)
    block = jnp.concatenate([
        _csum(xo), _csum(xo * xo), _csum(xc), _csum(xc * xc),
        _csum(xo * xc), jnp.zeros((3, xo.shape[1]), jnp.float32)
    ], axis=0)

    @pl.when(j == 0)
    def _():
        stats_ref[...] = block

    @pl.when(j > 0)
    def _():
        stats_ref[...] += block


def _main_kernel(inv_b,
                 xb_ref, stats_ref,
                 cg1_ref, cb1_ref, cw1_ref, cc1_ref,
                 og1_ref, ob1_ref, ow1_ref, oc1_ref,
                 rg1_ref, rb1_ref, rw1_ref, rc1_ref,
                 cg2_ref, cb2_ref, cw2_ref, cc2_ref,
                 og2_ref, ob2_ref, ow2_ref, oc2_ref,
                 rg2_ref, rb2_ref, rw2_ref, rc2_ref,
                 out_c_ref, out_o_ref, out_r_ref,
                 w1big_ref, b1big_ref, w2co_ref, w2r_ref, b2_ref,
                 hstats_ref):
    p = pl.program_id(0)
    j = pl.program_id(1)
    H = cw1_ref.shape[0]

    def fold(m, v, g_ref, b_ref, w_ref, c_ref):
        a = g_ref[...] * jax.lax.rsqrt(v + _EPS)
        ws = (jnp.transpose(a) * w_ref[...]).astype(jnp.bfloat16)
        bias = (jnp.dot(b_ref[...] - a * m, w_ref[...],
                        preferred_element_type=jnp.float32) + c_ref[...])
        return ws, bias

    @pl.when((p == 0) & (j == 0))
    def _():
        s = stats_ref[...]
        m_xo = s[0:1] * inv_b
        v_xo = s[1:2] * inv_b - m_xo * m_xo
        m_xc = s[2:3] * inv_b
        v_xc = s[3:4] * inv_b - m_xc * m_xc
        m_xr = m_xo + m_xc
        v_xr = (s[1:2] + s[3:4] + 2.0 * s[4:5]) * inv_b - m_xr * m_xr
        wc, bc = fold(m_xc, v_xc, cg1_ref, cb1_ref, cw1_ref, cc1_ref)
        wo, bo = fold(m_xo, v_xo, og1_ref, ob1_ref, ow1_ref, oc1_ref)
        wr, br = fold(m_xr, v_xr, rg1_ref, rb1_ref, rw1_ref, rc1_ref)
        z = jnp.zeros((H, H), jnp.bfloat16)
        w1big_ref[...] = jnp.concatenate([
            jnp.concatenate([wc, z, wr], axis=1),
            jnp.concatenate([z, wo, wr], axis=1)], axis=0)
        b1big_ref[...] = jnp.concatenate([bc, bo, br], axis=1)

    @pl.when((p == 1) & (j == 0))
    def _():
        hs = hstats_ref[...]
        folded = []
        for k, (g_ref, b_ref, w_ref, c_ref) in enumerate(
                ((cg2_ref, cb2_ref, cw2_ref, cc2_ref),
                 (og2_ref, ob2_ref, ow2_ref, oc2_ref),
                 (rg2_ref, rb2_ref, rw2_ref, rc2_ref))):
            m = hs[0:1, k * H:(k + 1) * H] * inv_b
            v = hs[1:2, k * H:(k + 1) * H] * inv_b - m * m
            folded.append(fold(m, v, g_ref, b_ref, w_ref, c_ref))
        (wsc, bc), (wso, bo), (wsr, br) = folded
        z = jnp.zeros((H, H), jnp.bfloat16)
        w2co_ref[...] = jnp.concatenate([
            jnp.concatenate([wsc, z], axis=1),
            jnp.concatenate([z, wso], axis=1)], axis=0)
        w2r_ref[...] = wsr
        b2_ref[0:1, :] = jnp.concatenate([bc, bo], axis=1)
        b2_ref[1:2, 0:H] = br

    n_sub = _NSUB
    rc = xb_ref.shape[0] // n_sub

    def hidden(c):
        xbig = xb_ref[c * rc:(c + 1) * rc, :]
        return jnp.maximum(
            jnp.dot(xbig, w1big_ref[...], preferred_element_type=jnp.float32)
            + b1big_ref[...], 0.0)

    @pl.when(p == 0)
    def _():
        tot = None
        for c in range(n_sub):
            h = hidden(c)
            part = jnp.concatenate([_csum(h), _csum(h * h)], axis=0)
            tot = part if tot is None else tot + part
        block = jnp.concatenate(
            [tot, jnp.zeros((6, tot.shape[1]), jnp.float32)], axis=0)

        @pl.when(j == 0)
        def _():
            hstats_ref[...] = block

        @pl.when(j > 0)
        def _():
            hstats_ref[...] += block

    @pl.when(p == 1)
    def _():
        def log_softmax(z):
            m = jnp.max(z, axis=-1, keepdims=True)
            s = z - m
            return s - jnp.log(jnp.sum(jnp.exp(s), axis=-1, keepdims=True))

        for c in range(n_sub):
            sl = slice(c * rc, (c + 1) * rc)
            hb = hidden(c).astype(jnp.bfloat16)
            z_co = (jnp.dot(hb[:, 0:2 * H], w2co_ref[...],
                            preferred_element_type=jnp.float32)
                    + b2_ref[0:1, :])
            z_r = (jnp.dot(hb[:, 2 * H:3 * H], w2r_ref[...],
                           preferred_element_type=jnp.float32)
                   + b2_ref[1:2, 0:H])
            out_c_ref[sl, :] = log_softmax(z_co[:, 0:H])
            out_o_ref[sl, :] = z_co[:, H:2 * H]
            out_r_ref[sl, :] = log_softmax(z_r)


def _row1(r, h):
    return pl.BlockSpec((r, h), lambda j: (j, 0))


def _vec2(h):
    return pl.BlockSpec((1, h), lambda p, j: (0, 0))


def _mat2(h, o):
    return pl.BlockSpec((h, o), lambda p, j: (0, 0))


@functools.partial(jax.jit, static_argnames=())
def kernel(xo, xc,
           ctx_g1, ctx_b1, ctx_W1, ctx_c1, ctx_g2, ctx_b2, ctx_W2, ctx_c2,
           obj_g1, obj_b1, obj_W1, obj_c1, obj_g2, obj_b2, obj_W2, obj_c2,
           rnd_g1, rnd_b1, rnd_W1, rnd_c1, rnd_g2, rnd_b2, rnd_W2, rnd_c2):
    B, H = xo.shape
    O = ctx_W2.shape[1]
    R = 10000 if B % 10000 == 0 else (1000 if B % 1000 == 0 else B)
    nb = B // R

    stats, xb = pl.pallas_call(
        _stage_kernel,
        grid=(nb,),
        in_specs=[_row1(R, H), _row1(R, H)],
        out_specs=[pl.BlockSpec((8, H), lambda j: (0, 0)),
                   _row1(R, 2 * H)],
        out_shape=[jax.ShapeDtypeStruct((8, H), jnp.float32),
                   jax.ShapeDtypeStruct((B, 2 * H), jnp.bfloat16)],
    )(xo, xc)

    vecs = {k: v.reshape(1, H) for k, v in dict(
        cg1=ctx_g1, cb1=ctx_b1, cc1=ctx_c1, og1=obj_g1, ob1=obj_b1,
        oc1=obj_c1, rg1=rnd_g1, rb1=rnd_b1, rc1=rnd_c1,
        cg2=ctx_g2, cb2=ctx_b2, cc2=ctx_c2, og2=obj_g2, ob2=obj_b2,
        oc2=obj_c2, rg2=rnd_g2, rb2=rnd_b2, rc2=rnd_c2).items()}

    Rb = 10000 if B % 10000 == 0 else R
    nbb = B // Rb
    row_in = pl.BlockSpec((Rb, 2 * H), lambda p, j: (j, 0))
    row_out = pl.BlockSpec((Rb, O), lambda p, j: (p * j, 0))

    outs = pl.pallas_call(
        functools.partial(_main_kernel, 1.0 / B),
        grid=(2, nbb),
        in_specs=[row_in, pl.BlockSpec((8, H), lambda p, j: (0, 0)),
                  _vec2(H), _vec2(H), _mat2(H, H), _vec2(H),
                  _vec2(H), _vec2(H), _mat2(H, H), _vec2(H),
                  _vec2(H), _vec2(H), _mat2(H, H), _vec2(H),
                  _vec2(H), _vec2(H), _mat2(H, O), _vec2(O),
                  _vec2(H), _vec2(H), _mat2(H, O), _vec2(O),
                  _vec2(H), _vec2(H), _mat2(H, O), _vec2(O)],
        out_specs=[row_out, row_out, row_out],
        out_shape=[jax.ShapeDtypeStruct((B, O), jnp.float32)] * 3,
        scratch_shapes=[pltpu.VMEM((2 * H, 3 * H), jnp.bfloat16),
                        pltpu.VMEM((1, 3 * H), jnp.float32),
                        pltpu.VMEM((2 * H, 2 * H), jnp.bfloat16),
                        pltpu.VMEM((H, O), jnp.bfloat16),
                        pltpu.VMEM((8, 2 * H), jnp.float32),
                        pltpu.VMEM((8, 3 * H), jnp.float32)],
    )(xb, stats,
      vecs["cg1"], vecs["cb1"], ctx_W1, vecs["cc1"],
      vecs["og1"], vecs["ob1"], obj_W1, vecs["oc1"],
      vecs["rg1"], vecs["rb1"], rnd_W1, vecs["rc1"],
      vecs["cg2"], vecs["cb2"], ctx_W2, vecs["cc2"],
      vecs["og2"], vecs["ob2"], obj_W2, vecs["oc2"],
      vecs["rg2"], vecs["rb2"], rnd_W2, vecs["rc2"])

    return tuple(outs)


# speedup vs baseline: 3.2276x; 1.0179x over previous
import functools

import jax
import jax.numpy as jnp
from jax.experimental import pallas as pl
from jax.experimental.pallas import tpu as pltpu

_EPS = 1e-5
_NSUB = 4


def _csum(x):
    return jnp.sum(x, axis=0, keepdims=True)


def _stage_kernel(xo_ref, xc_ref, stats_ref, xb_ref):
    j = pl.program_id(0)
    xo = xo_ref[...]
    xc = xc_ref[...]
    xb_ref[...] = jnp.concatenate(
        [xc.astype(jnp.bfloat16), xo.astype(jnp.bfloat16)], axis=1)
    block = jnp.concatenate([
        _csum(xo), _csum(xo * xo), _csum(xc), _csum(xc * xc),
        _csum(xo * xc), jnp.zeros((3, xo.shape[1]), jnp.float32)
    ], axis=0)

    @pl.when(j == 0)
    def _():
        stats_ref[...] = block

    @pl.when(j > 0)
    def _():
        stats_ref[...] += block


def _main_kernel(inv_b,
                 xb_ref, stats_ref,
                 cg1_ref, cb1_ref, cw1_ref, cc1_ref,
                 og1_ref, ob1_ref, ow1_ref, oc1_ref,
                 rg1_ref, rb1_ref, rw1_ref, rc1_ref,
                 cg2_ref, cb2_ref, cw2_ref, cc2_ref,
                 og2_ref, ob2_ref, ow2_ref, oc2_ref,
                 rg2_ref, rb2_ref, rw2_ref, rc2_ref,
                 out_c_ref, out_o_ref, out_r_ref,
                 w1big_ref, b1big_ref, w2co_ref, w2r_ref, b2_ref,
                 hstats_ref):
    p = pl.program_id(0)
    j = pl.program_id(1)
    H = cw1_ref.shape[0]

    def fold(m, v, g_ref, b_ref, w_ref, c_ref):
        a = g_ref[...] * jax.lax.rsqrt(v + _EPS)
        ws = (jnp.transpose(a) * w_ref[...]).astype(jnp.bfloat16)
        bias = (jnp.dot(b_ref[...] - a * m, w_ref[...],
                        preferred_element_type=jnp.float32) + c_ref[...])
        return ws, bias

    @pl.when((p == 0) & (j == 0))
    def _():
        s = stats_ref[...]
        m_xo = s[0:1] * inv_b
        v_xo = s[1:2] * inv_b - m_xo * m_xo
        m_xc = s[2:3] * inv_b
        v_xc = s[3:4] * inv_b - m_xc * m_xc
        m_xr = m_xo + m_xc
        v_xr = (s[1:2] + s[3:4] + 2.0 * s[4:5]) * inv_b - m_xr * m_xr
        wc, bc = fold(m_xc, v_xc, cg1_ref, cb1_ref, cw1_ref, cc1_ref)
        wo, bo = fold(m_xo, v_xo, og1_ref, ob1_ref, ow1_ref, oc1_ref)
        wr, br = fold(m_xr, v_xr, rg1_ref, rb1_ref, rw1_ref, rc1_ref)
        z = jnp.zeros((H, H), jnp.bfloat16)
        w1big_ref[...] = jnp.concatenate([
            jnp.concatenate([wc, z, wr], axis=1),
            jnp.concatenate([z, wo, wr], axis=1)], axis=0)
        b1big_ref[...] = jnp.concatenate([bc, bo, br], axis=1)

    @pl.when((p == 1) & (j == 0))
    def _():
        hs = hstats_ref[...]
        folded = []
        for k, (g_ref, b_ref, w_ref, c_ref) in enumerate(
                ((cg2_ref, cb2_ref, cw2_ref, cc2_ref),
                 (og2_ref, ob2_ref, ow2_ref, oc2_ref),
                 (rg2_ref, rb2_ref, rw2_ref, rc2_ref))):
            m = hs[0:1, k * H:(k + 1) * H] * inv_b
            v = hs[1:2, k * H:(k + 1) * H] * inv_b - m * m
            folded.append(fold(m, v, g_ref, b_ref, w_ref, c_ref))
        (wsc, bc), (wso, bo), (wsr, br) = folded
        z = jnp.zeros((H, H), jnp.bfloat16)
        w2co_ref[...] = jnp.concatenate([
            jnp.concatenate([wsc, z], axis=1),
            jnp.concatenate([z, wso], axis=1)], axis=0)
        w2r_ref[...] = wsr
        b2_ref[0:1, :] = jnp.concatenate([bc, bo], axis=1)
        b2_ref[1:2, 0:H] = br

    n_sub = _NSUB
    rc = xb_ref.shape[0] // n_sub

    def hidden(c):
        xbig = xb_ref[c * rc:(c + 1) * rc, :]
        return jnp.maximum(
            jnp.dot(xbig, w1big_ref[...], preferred_element_type=jnp.float32)
            + b1big_ref[...], 0.0)

    @pl.when(p == 0)
    def _():
        tot = None
        for c in range(n_sub):
            h = hidden(c)
            part = jnp.concatenate([_csum(h), _csum(h * h)], axis=0)
            tot = part if tot is None else tot + part
        block = jnp.concatenate(
            [tot, jnp.zeros((6, tot.shape[1]), jnp.float32)], axis=0)

        @pl.when(j == 0)
        def _():
            hstats_ref[...] = block

        @pl.when(j > 0)
        def _():
            hstats_ref[...] += block

    @pl.when(p == 1)
    def _():
        def log_softmax(z):
            m = jnp.max(z, axis=-1, keepdims=True)
            s = z - m
            return s - jnp.log(jnp.sum(jnp.exp(s), axis=-1, keepdims=True))

        for c in range(n_sub):
            sl = slice(c * rc, (c + 1) * rc)
            hb = hidden(c).astype(jnp.bfloat16)
            z_co = (jnp.dot(hb[:, 0:2 * H], w2co_ref[...],
                            preferred_element_type=jnp.float32)
                    + b2_ref[0:1, :])
            z_r = (jnp.dot(hb[:, 2 * H:3 * H], w2r_ref[...],
                           preferred_element_type=jnp.float32)
                   + b2_ref[1:2, 0:H])
            out_c_ref[sl, :] = log_softmax(z_co[:, 0:H])
            out_o_ref[sl, :] = z_co[:, H:2 * H]
            out_r_ref[sl, :] = log_softmax(z_r)


def _row1(r, h):
    return pl.BlockSpec((r, h), lambda j: (j, 0))


def _vec2(h):
    return pl.BlockSpec((1, h), lambda p, j: (0, 0))


def _mat2(h, o):
    return pl.BlockSpec((h, o), lambda p, j: (0, 0))


@functools.partial(jax.jit, static_argnames=())
def kernel(xo, xc,
           ctx_g1, ctx_b1, ctx_W1, ctx_c1, ctx_g2, ctx_b2, ctx_W2, ctx_c2,
           obj_g1, obj_b1, obj_W1, obj_c1, obj_g2, obj_b2, obj_W2, obj_c2,
           rnd_g1, rnd_b1, rnd_W1, rnd_c1, rnd_g2, rnd_b2, rnd_W2, rnd_c2):
    B, H = xo.shape
    O = ctx_W2.shape[1]
    R = 10000 if B % 10000 == 0 else (1000 if B % 1000 == 0 else B)
    nb = B // R

    stats, xb = pl.pallas_call(
        _stage_kernel,
        grid=(nb,),
        in_specs=[_row1(R, H), _row1(R, H)],
        out_specs=[pl.BlockSpec((8, H), lambda j: (0, 0)),
                   _row1(R, 2 * H)],
        out_shape=[jax.ShapeDtypeStruct((8, H), jnp.float32),
                   jax.ShapeDtypeStruct((B, 2 * H), jnp.bfloat16)],
    )(xo, xc)

    vecs = {k: v.reshape(1, H) for k, v in dict(
        cg1=ctx_g1, cb1=ctx_b1, cc1=ctx_c1, og1=obj_g1, ob1=obj_b1,
        oc1=obj_c1, rg1=rnd_g1, rb1=rnd_b1, rc1=rnd_c1,
        cg2=ctx_g2, cb2=ctx_b2, cc2=ctx_c2, og2=obj_g2, ob2=obj_b2,
        oc2=obj_c2, rg2=rnd_g2, rb2=rnd_b2, rc2=rnd_c2).items()}

    Rb = 10000 if B % 10000 == 0 else R
    nbb = B // Rb
    row_in = pl.BlockSpec((Rb, 2 * H), lambda p, j: (j, 0))
    row_out = pl.BlockSpec((Rb, O), lambda p, j: (p * j, 0))

    outs = pl.pallas_call(
        functools.partial(_main_kernel, 1.0 / B),
        grid=(2, nbb),
        in_specs=[row_in, pl.BlockSpec((8, H), lambda p, j: (0, 0)),
                  _vec2(H), _vec2(H), _mat2(H, H), _vec2(H),
                  _vec2(H), _vec2(H), _mat2(H, H), _vec2(H),
                  _vec2(H), _vec2(H), _mat2(H, H), _vec2(H),
                  _vec2(H), _vec2(H), _mat2(H, O), _vec2(O),
                  _vec2(H), _vec2(H), _mat2(H, O), _vec2(O),
                  _vec2(H), _vec2(H), _mat2(H, O), _vec2(O)],
        out_specs=[row_out, row_out, row_out],
        out_shape=[jax.ShapeDtypeStruct((B, O), jnp.float32)] * 3,
        scratch_shapes=[pltpu.VMEM((2 * H, 3 * H), jnp.bfloat16),
                        pltpu.VMEM((1, 3 * H), jnp.float32),
                        pltpu.VMEM((2 * H, 2 * H), jnp.bfloat16),
                        pltpu.VMEM((H, O), jnp.bfloat16),
                        pltpu.VMEM((8, 2 * H), jnp.float32),
                        pltpu.VMEM((8, 3 * H), jnp.float32)],
    )(xb, stats,
      vecs["cg1"], vecs["cb1"], ctx_W1, vecs["cc1"],
      vecs["og1"], vecs["ob1"], obj_W1, vecs["oc1"],
      vecs["rg1"], vecs["rb1"], rnd_W1, vecs["rc1"],
      vecs["cg2"], vecs["cb2"], ctx_W2, vecs["cc2"],
      vecs["og2"], vecs["ob2"], obj_W2, vecs["oc2"],
      vecs["rg2"], vecs["rb2"], rnd_W2, vecs["rc2"])

    return tuple(outs)


# speedup vs baseline: 3.2347x; 1.0022x over previous
import functools

import jax
import jax.numpy as jnp
from jax.experimental import pallas as pl
from jax.experimental.pallas import tpu as pltpu

_EPS = 1e-5
_NSUB = 8


def _csum(x):
    return jnp.sum(x, axis=0, keepdims=True)


def _stage_kernel(xo_ref, xc_ref, stats_ref, xb_ref):
    j = pl.program_id(0)
    rc = xo_ref.shape[0] // _NSUB
    tot = None
    for c in range(_NSUB):
        sl = slice(c * rc, (c + 1) * rc)
        xo = xo_ref[sl, :]
        xc = xc_ref[sl, :]
        xb_ref[sl, :] = jnp.concatenate(
            [xc.astype(jnp.bfloat16), xo.astype(jnp.bfloat16)], axis=1)
        part = jnp.concatenate([
            _csum(xo), _csum(xo * xo), _csum(xc), _csum(xc * xc),
            _csum(xo * xc)], axis=0)
        tot = part if tot is None else tot + part
    block = jnp.concatenate(
        [tot, jnp.zeros((3, tot.shape[1]), jnp.float32)], axis=0)

    @pl.when(j == 0)
    def _():
        stats_ref[...] = block

    @pl.when(j > 0)
    def _():
        stats_ref[...] += block


def _main_kernel(inv_b,
                 xb_ref, stats_ref,
                 cg1_ref, cb1_ref, cw1_ref, cc1_ref,
                 og1_ref, ob1_ref, ow1_ref, oc1_ref,
                 rg1_ref, rb1_ref, rw1_ref, rc1_ref,
                 cg2_ref, cb2_ref, cw2_ref, cc2_ref,
                 og2_ref, ob2_ref, ow2_ref, oc2_ref,
                 rg2_ref, rb2_ref, rw2_ref, rc2_ref,
                 out_c_ref, out_o_ref, out_r_ref,
                 w1big_ref, b1big_ref, w2co_ref, w2r_ref, b2_ref,
                 hstats_ref):
    p = pl.program_id(0)
    j = pl.program_id(1)
    H = cw1_ref.shape[0]

    def fold(m, v, g_ref, b_ref, w_ref, c_ref):
        a = g_ref[...] * jax.lax.rsqrt(v + _EPS)
        ws = (jnp.transpose(a) * w_ref[...]).astype(jnp.bfloat16)
        bias = (jnp.dot(b_ref[...] - a * m, w_ref[...],
                        preferred_element_type=jnp.float32) + c_ref[...])
        return ws, bias

    @pl.when((p == 0) & (j == 0))
    def _():
        s = stats_ref[...]
        m_xo = s[0:1] * inv_b
        v_xo = s[1:2] * inv_b - m_xo * m_xo
        m_xc = s[2:3] * inv_b
        v_xc = s[3:4] * inv_b - m_xc * m_xc
        m_xr = m_xo + m_xc
        v_xr = (s[1:2] + s[3:4] + 2.0 * s[4:5]) * inv_b - m_xr * m_xr
        wc, bc = fold(m_xc, v_xc, cg1_ref, cb1_ref, cw1_ref, cc1_ref)
        wo, bo = fold(m_xo, v_xo, og1_ref, ob1_ref, ow1_ref, oc1_ref)
        wr, br = fold(m_xr, v_xr, rg1_ref, rb1_ref, rw1_ref, rc1_ref)
        z = jnp.zeros((H, H), jnp.bfloat16)
        w1big_ref[...] = jnp.concatenate([
            jnp.concatenate([wc, z, wr], axis=1),
            jnp.concatenate([z, wo, wr], axis=1)], axis=0)
        b1big_ref[...] = jnp.concatenate([bc, bo, br], axis=1)

    @pl.when((p == 1) & (j == 0))
    def _():
        hs = hstats_ref[...]
        folded = []
        for k, (g_ref, b_ref, w_ref, c_ref) in enumerate(
                ((cg2_ref, cb2_ref, cw2_ref, cc2_ref),
                 (og2_ref, ob2_ref, ow2_ref, oc2_ref),
                 (rg2_ref, rb2_ref, rw2_ref, rc2_ref))):
            m = hs[0:1, k * H:(k + 1) * H] * inv_b
            v = hs[1:2, k * H:(k + 1) * H] * inv_b - m * m
            folded.append(fold(m, v, g_ref, b_ref, w_ref, c_ref))
        (wsc, bc), (wso, bo), (wsr, br) = folded
        z = jnp.zeros((H, H), jnp.bfloat16)
        w2co_ref[...] = jnp.concatenate([
            jnp.concatenate([wsc, z], axis=1),
            jnp.concatenate([z, wso], axis=1)], axis=0)
        w2r_ref[...] = wsr
        b2_ref[0:1, :] = jnp.concatenate([bc, bo], axis=1)
        b2_ref[1:2, 0:H] = br

    n_sub = _NSUB
    rc = xb_ref.shape[0] // n_sub

    def hidden(c):
        xbig = xb_ref[c * rc:(c + 1) * rc, :]
        return jnp.maximum(
            jnp.dot(xbig, w1big_ref[...], preferred_element_type=jnp.float32)
            + b1big_ref[...], 0.0)

    @pl.when(p == 0)
    def _():
        tot = None
        for c in range(n_sub):
            h = hidden(c)
            part = jnp.concatenate([_csum(h), _csum(h * h)], axis=0)
            tot = part if tot is None else tot + part
        block = jnp.concatenate(
            [tot, jnp.zeros((6, tot.shape[1]), jnp.float32)], axis=0)

        @pl.when(j == 0)
        def _():
            hstats_ref[...] = block

        @pl.when(j > 0)
        def _():
            hstats_ref[...] += block

    @pl.when(p == 1)
    def _():
        def log_softmax(z):
            m = jnp.max(z, axis=-1, keepdims=True)
            s = z - m
            return s - jnp.log(jnp.sum(jnp.exp(s), axis=-1, keepdims=True))

        for c in range(n_sub):
            sl = slice(c * rc, (c + 1) * rc)
            hb = hidden(c).astype(jnp.bfloat16)
            z_co = (jnp.dot(hb[:, 0:2 * H], w2co_ref[...],
                            preferred_element_type=jnp.float32)
                    + b2_ref[0:1, :])
            z_r = (jnp.dot(hb[:, 2 * H:3 * H], w2r_ref[...],
                           preferred_element_type=jnp.float32)
                   + b2_ref[1:2, 0:H])
            out_c_ref[sl, :] = log_softmax(z_co[:, 0:H])
            out_o_ref[sl, :] = z_co[:, H:2 * H]
            out_r_ref[sl, :] = log_softmax(z_r)


def _row1(r, h):
    return pl.BlockSpec((r, h), lambda j: (j, 0))


def _vec2(h):
    return pl.BlockSpec((1, h), lambda p, j: (0, 0))


def _mat2(h, o):
    return pl.BlockSpec((h, o), lambda p, j: (0, 0))


@functools.partial(jax.jit, static_argnames=())
def kernel(xo, xc,
           ctx_g1, ctx_b1, ctx_W1, ctx_c1, ctx_g2, ctx_b2, ctx_W2, ctx_c2,
           obj_g1, obj_b1, obj_W1, obj_c1, obj_g2, obj_b2, obj_W2, obj_c2,
           rnd_g1, rnd_b1, rnd_W1, rnd_c1, rnd_g2, rnd_b2, rnd_W2, rnd_c2):
    B, H = xo.shape
    O = ctx_W2.shape[1]
    R = 10000 if B % 10000 == 0 else (1000 if B % 1000 == 0 else B)
    nb = B // R

    stats, xb = pl.pallas_call(
        _stage_kernel,
        grid=(nb,),
        in_specs=[_row1(R, H), _row1(R, H)],
        out_specs=[pl.BlockSpec((8, H), lambda j: (0, 0)),
                   _row1(R, 2 * H)],
        out_shape=[jax.ShapeDtypeStruct((8, H), jnp.float32),
                   jax.ShapeDtypeStruct((B, 2 * H), jnp.bfloat16)],
    )(xo, xc)

    vecs = {k: v.reshape(1, H) for k, v in dict(
        cg1=ctx_g1, cb1=ctx_b1, cc1=ctx_c1, og1=obj_g1, ob1=obj_b1,
        oc1=obj_c1, rg1=rnd_g1, rb1=rnd_b1, rc1=rnd_c1,
        cg2=ctx_g2, cb2=ctx_b2, cc2=ctx_c2, og2=obj_g2, ob2=obj_b2,
        oc2=obj_c2, rg2=rnd_g2, rb2=rnd_b2, rc2=rnd_c2).items()}

    Rb = 10000 if B % 10000 == 0 else R
    nbb = B // Rb
    row_in = pl.BlockSpec((Rb, 2 * H), lambda p, j: (j, 0))
    row_out = pl.BlockSpec((Rb, O), lambda p, j: (p * j, 0))

    outs = pl.pallas_call(
        functools.partial(_main_kernel, 1.0 / B),
        grid=(2, nbb),
        in_specs=[row_in, pl.BlockSpec((8, H), lambda p, j: (0, 0)),
                  _vec2(H), _vec2(H), _mat2(H, H), _vec2(H),
                  _vec2(H), _vec2(H), _mat2(H, H), _vec2(H),
                  _vec2(H), _vec2(H), _mat2(H, H), _vec2(H),
                  _vec2(H), _vec2(H), _mat2(H, O), _vec2(O),
                  _vec2(H), _vec2(H), _mat2(H, O), _vec2(O),
                  _vec2(H), _vec2(H), _mat2(H, O), _vec2(O)],
        out_specs=[row_out, row_out, row_out],
        out_shape=[jax.ShapeDtypeStruct((B, O), jnp.float32)] * 3,
        scratch_shapes=[pltpu.VMEM((2 * H, 3 * H), jnp.bfloat16),
                        pltpu.VMEM((1, 3 * H), jnp.float32),
                        pltpu.VMEM((2 * H, 2 * H), jnp.bfloat16),
                        pltpu.VMEM((H, O), jnp.bfloat16),
                        pltpu.VMEM((8, 2 * H), jnp.float32),
                        pltpu.VMEM((8, 3 * H), jnp.float32)],
    )(xb, stats,
      vecs["cg1"], vecs["cb1"], ctx_W1, vecs["cc1"],
      vecs["og1"], vecs["ob1"], obj_W1, vecs["oc1"],
      vecs["rg1"], vecs["rb1"], rnd_W1, vecs["rc1"],
      vecs["cg2"], vecs["cb2"], ctx_W2, vecs["cc2"],
      vecs["og2"], vecs["ob2"], obj_W2, vecs["oc2"],
      vecs["rg2"], vecs["rb2"], rnd_W2, vecs["rc2"])

    return tuple(outs)
